# Initial kernel scaffold; baseline (speedup 1.0000x reference)
import jax
import jax.numpy as jnp
from jax.experimental import pallas as pl


def kernel(emb_e, token_pos, emb_table, distribution):
    raise NotImplementedError("write your pallas kernel here")


# speedup vs baseline: 9031.3611x; 9031.3611x over previous
import functools

import jax
import jax.numpy as jnp
from jax import lax
from jax.experimental import pallas as pl
from jax.experimental.pallas import tpu as pltpu
from jax.experimental.pallas import tpu_sc as plsc

N_NEG = 5
VOCAB = 1_000_000
DIM = 64
BSZ = 1024
PAD_LEN = 200

VPAD = 1 << 20
NT = 32
CHUNK_V = VPAD // NT
L = 16
G = 32
NCOARSE = VPAD // G
NNEG = N_NEG * PAD_LEN
NNEG_P = 1008
POS_P = 208
TOT_P = POS_P + NNEG_P
SAMP_PT = BSZ * NNEG_P // NT
SUB = 128
BATCH = 6 * SUB
NBATCH = SAMP_PT // BATCH
ROWS_PT = BSZ // NT
NEG_MASK_PAD = -30.0


def _mesh():
    return plsc.VectorSubcoreMesh(core_axis_name="c", subcore_axis_name="s",
                                  num_cores=2, num_subcores=16)


def _wid():
    return lax.axis_index("s") * 2 + lax.axis_index("c")


def _iota():
    return lax.iota(jnp.int32, 16)


def _k1_uniforms():
    def body(o_ref):
        pltpu.prng_seed(42)
        bits = pltpu.prng_random_bits((BSZ, NNEG_P))
        v = (jax.lax.shift_right_logical(bits.astype(jnp.uint32),
                                         jnp.uint32(8))
             ).astype(jnp.float32) * (2.0 ** -24)
        col = lax.broadcasted_iota(jnp.int32, (BSZ, NNEG_P), 1)
        u = jnp.where(col < NNEG, (col.astype(jnp.float32) + v) * (1.0 / NNEG),
                      0.0)
        o_ref[...] = u

    return pl.pallas_call(
        body, out_shape=jax.ShapeDtypeStruct((BSZ, NNEG_P), jnp.float32))()


def _k2_sums(dist_p):
    @functools.partial(
        pl.kernel,
        out_type=jax.ShapeDtypeStruct((NT, L), jnp.float32),
        mesh=_mesh(),
        compiler_params=pltpu.CompilerParams(needs_layout_passes=False, use_tc_tiling_on_sc=False),
        scratch_types=[pltpu.VMEM((CHUNK_V,), jnp.float32),
                       pltpu.VMEM((L,), jnp.float32)],
    )
    def k(dist_hbm, out_hbm, dvmem, svmem):
        wid = _wid()
        pltpu.sync_copy(dist_hbm.at[pl.ds(wid * CHUNK_V, CHUNK_V)], dvmem)

        def body(i, acc):
            return acc + dvmem[pl.ds(i * L, L)]

        acc = lax.fori_loop(0, CHUNK_V // L, body,
                            jnp.zeros((L,), jnp.float32))
        svmem[...] = jnp.broadcast_to(jnp.sum(acc), (L,))
        pltpu.sync_copy(svmem, out_hbm.at[wid])

    return k(dist_p)


def _k3_cdf(dist_p, tsums):
    @functools.partial(
        pl.kernel,
        out_type=(jax.ShapeDtypeStruct((VPAD,), jnp.float32),
                  jax.ShapeDtypeStruct((NCOARSE,), jnp.float32)),
        mesh=_mesh(),
        compiler_params=pltpu.CompilerParams(needs_layout_passes=False, use_tc_tiling_on_sc=False),
        scratch_types=[pltpu.VMEM((CHUNK_V,), jnp.float32),
                       pltpu.VMEM((CHUNK_V,), jnp.float32),
                       pltpu.VMEM((NT, L), jnp.float32),
                       pltpu.VMEM((CHUNK_V // G,), jnp.float32)],
    )
    def k(dist_hbm, tsums_hbm, cdf_hbm, coarse_hbm, dvmem, cvmem, tsvmem,
          crsvmem):
        wid = _wid()
        pltpu.sync_copy(tsums_hbm, tsvmem)
        pltpu.sync_copy(dist_hbm.at[pl.ds(wid * CHUNK_V, CHUNK_V)], dvmem)

        base = jnp.float32(0.0)
        for w2 in range(NT):
            s = jnp.max(tsvmem[w2])
            base = base + jnp.where(w2 < wid, s, 0.0)

        def cum_body(i, carry):
            v = dvmem[pl.ds(i * L, L)]
            cvmem[pl.ds(i * L, L)] = plsc.cumsum(v) + carry
            return carry + jnp.sum(v)

        lax.fori_loop(0, CHUNK_V // L, cum_body, base)

        iota = _iota()

        def crs_body(j, _):
            idx = (j * L + iota) * G + (G - 1)
            crsvmem[pl.ds(j * L, L)] = plsc.load_gather(cvmem, [idx])
            return 0

        lax.fori_loop(0, CHUNK_V // G // L, crs_body, 0)

        pltpu.sync_copy(cvmem, cdf_hbm.at[pl.ds(wid * CHUNK_V, CHUNK_V)])
        pltpu.sync_copy(
            crsvmem,
            coarse_hbm.at[pl.ds(wid * (CHUNK_V // G), CHUNK_V // G)])

    return k(dist_p, tsums)


def _k4_sample(coarse, cdf2d, u_flat):
    @functools.partial(
        pl.kernel,
        out_type=jax.ShapeDtypeStruct((BSZ * NNEG_P,), jnp.int32),
        mesh=_mesh(),
        compiler_params=pltpu.CompilerParams(needs_layout_passes=False, use_tc_tiling_on_sc=False),
        scratch_types=[pltpu.VMEM((NCOARSE,), jnp.float32),
                       pltpu.VMEM((BATCH,), jnp.float32),
                       pltpu.VMEM((BATCH,), jnp.int32),
                       pltpu.VMEM((BATCH, G), jnp.float32),
                       pltpu.VMEM((BATCH,), jnp.int32),
                       pltpu.SemaphoreType.DMA],
    )
    def k2(coarse_hbm, cdf2_hbm, u_hbm, tok_hbm, cvmem, uvmem, bidvmem,
           blkvmem, tokvmem, sem):
        wid = _wid()
        pltpu.sync_copy(coarse_hbm, cvmem)
        total = jnp.max(cvmem[pl.ds(NCOARSE - L, L)])
        iota = _iota()

        def batch_body(bi, _):
            base = wid * SAMP_PT + bi * BATCH
            pltpu.sync_copy(u_hbm.at[pl.ds(base, BATCH)], uvmem)

            def coarse_body(v, _):
                u = uvmem[pl.ds(v * L, L)] * total
                lo = jnp.zeros((L,), jnp.int32)
                hi = jnp.full((L,), NCOARSE - 1, jnp.int32)
                for _s in range(15):
                    mid = jax.lax.shift_right_logical(lo + hi, 1)
                    c = plsc.load_gather(cvmem, [mid])
                    pred = u < c
                    hi = jnp.where(pred, mid, hi)
                    lo = jnp.where(pred, lo, mid + 1)
                bidvmem[pl.ds(v * L, L)] = jnp.minimum(lo, NCOARSE - 1)
                return 0

            lax.fori_loop(0, BATCH // L, coarse_body, 0)

            copies = []
            for s in range(BATCH // SUB):
                copies.append(pltpu.async_copy(
                    cdf2_hbm.at[bidvmem.at[pl.ds(s * SUB, SUB)]],
                    blkvmem.at[pl.ds(s * SUB, SUB)], sem))
            for c in copies:
                c.wait()

            def fine_body(v, _):
                u = uvmem[pl.ds(v * L, L)] * total
                row = v * L + iota
                blk = bidvmem[pl.ds(v * L, L)]
                lo = jnp.zeros((L,), jnp.int32)
                hi = jnp.full((L,), G - 1, jnp.int32)
                for _s in range(5):
                    mid = jax.lax.shift_right_logical(lo + hi, 1)
                    c = plsc.load_gather(blkvmem, [row, mid])
                    pred = u < c
                    hi = jnp.where(pred, mid, hi)
                    lo = jnp.where(pred, lo, mid + 1)
                tok = jnp.minimum(blk * G + lo, VOCAB - 1)
                tokvmem[pl.ds(v * L, L)] = tok
                return 0

            lax.fori_loop(0, BATCH // L, fine_body, 0)
            pltpu.sync_copy(tokvmem, tok_hbm.at[pl.ds(base, BATCH)])
            return 0

        lax.fori_loop(0, NBATCH, batch_body, 0)

    return k2(coarse, cdf2d, u_flat)


def _k5_dots(emb_table, emb_e, tok_pos_p, tok_neg):
    @functools.partial(
        pl.kernel,
        out_type=jax.ShapeDtypeStruct((BSZ, TOT_P), jnp.float32),
        mesh=_mesh(),
        compiler_params=pltpu.CompilerParams(needs_layout_passes=False, use_tc_tiling_on_sc=False),
        scratch_types=[pltpu.VMEM((DIM,), jnp.float32),
                       pltpu.VMEM((POS_P,), jnp.int32),
                       pltpu.VMEM((NNEG_P,), jnp.int32),
                       pltpu.VMEM((TOT_P, DIM), jnp.float32),
                       pltpu.VMEM((TOT_P,), jnp.float32),
                       pltpu.SemaphoreType.DMA],
    )
    def k(table_hbm, embe_hbm, tpos_hbm, tneg_hbm, dots_hbm, evmem, pidvmem,
          nidvmem, rowsvmem, dotsvmem, sem):
        wid = _wid()
        iota = _iota()

        def row_body(bl, _):
            b = wid * ROWS_PT + bl
            pltpu.sync_copy(embe_hbm.at[b], evmem)
            pltpu.sync_copy(tpos_hbm.at[b], pidvmem)
            pltpu.sync_copy(tneg_hbm.at[b], nidvmem)
            copies = []
            for s in range(POS_P // 104):
                copies.append(pltpu.async_copy(
                    table_hbm.at[pidvmem.at[pl.ds(s * 104, 104)]],
                    rowsvmem.at[pl.ds(s * 104, 104)], sem))
            for s in range(NNEG_P // 112):
                copies.append(pltpu.async_copy(
                    table_hbm.at[nidvmem.at[pl.ds(s * 112, 112)]],
                    rowsvmem.at[pl.ds(POS_P + s * 112, 112)], sem))
            for c in copies:
                c.wait()

            evecs = [evmem[pl.ds(kk * L, L)] for kk in range(DIM // L)]
            es = [evecs[d // L][d % L] for d in range(DIM)]

            def pos_body(i, _):
                idx0 = i * L + iota
                acc = jnp.zeros((L,), jnp.float32)
                for d in range(DIM):
                    g = plsc.load_gather(
                        rowsvmem, [idx0, jnp.full((L,), d, jnp.int32)])
                    acc = acc - g * es[d]
                valid = (i * L + iota) < PAD_LEN
                dotsvmem[pl.ds(i * L, L)] = jnp.where(valid, acc,
                                                      NEG_MASK_PAD)
                return 0

            lax.fori_loop(0, POS_P // L, pos_body, 0)

            def neg_body(j, _):
                idx0 = POS_P + j * L + iota
                acc = jnp.zeros((L,), jnp.float32)
                for d in range(DIM):
                    g = plsc.load_gather(
                        rowsvmem, [idx0, jnp.full((L,), d, jnp.int32)])
                    acc = acc + g * es[d]
                valid = (j * L + iota) < NNEG
                dotsvmem[pl.ds(POS_P + j * L, L)] = jnp.where(
                    valid, acc, NEG_MASK_PAD)
                return 0

            lax.fori_loop(0, NNEG_P // L, neg_body, 0)
            pltpu.sync_copy(dotsvmem, dots_hbm.at[b])
            return 0

        lax.fori_loop(0, ROWS_PT, row_body, 0)

    return k(emb_table, emb_e, tok_pos_p, tok_neg)


def _k6_loss(dots, token_pos):
    def body(d_ref, t_ref, o_ref):
        d = d_ref[...]
        sp = jnp.maximum(d, 0.0) + jnp.log1p(jnp.exp(-jnp.abs(d)))
        cnt = jnp.sum((t_ref[...] != 0).astype(jnp.float32))
        o_ref[...] = (jnp.sum(sp) / (cnt * (N_NEG + 1)))[None, None]

    return pl.pallas_call(
        body, out_shape=jax.ShapeDtypeStruct((1, 1), jnp.float32))(
            dots, token_pos)


def kernel(emb_e, token_pos, emb_table, distribution):
    dist_p = jnp.concatenate(
        [distribution, jnp.zeros((VPAD - VOCAB,), jnp.float32)])
    tok_pos_p = jnp.pad(token_pos, ((0, 0), (0, POS_P - PAD_LEN)))
    u = _k1_uniforms()
    tsums = _k2_sums(dist_p)
    cdf, coarse = _k3_cdf(dist_p, tsums)
    tokens = _k4_sample(coarse, cdf.reshape(NCOARSE, G), u.reshape(-1))
    dots = _k5_dots(emb_table, emb_e, tok_pos_p,
                    tokens.reshape(BSZ, NNEG_P))
    out = _k6_loss(dots, token_pos)
    return out[0, 0]
